# baseline (speedup 1.0000x reference)
import jax
import jax.numpy as jnp
from jax.experimental import pallas as pl


_BL = 256


def _body(x_ref, pe_ref, o_ref):
    o_ref[...] = x_ref[...] + pe_ref[...]


def kernel(x, pe):
    b, l, d = x.shape
    nl = l // _BL
    return pl.pallas_call(
        _body,
        grid=(nl, b),
        in_specs=[
            pl.BlockSpec((1, _BL, d), lambda i, j: (j, i, 0)),
            pl.BlockSpec((_BL, d), lambda i, j: (i, 0)),
        ],
        out_specs=pl.BlockSpec((1, _BL, d), lambda i, j: (j, i, 0)),
        out_shape=jax.ShapeDtypeStruct((b, l, d), x.dtype),
    )(x, pe)


# speedup vs baseline: 1.4440x; 1.4440x over previous
import jax
import jax.numpy as jnp
from jax.experimental import pallas as pl


_BL = 1024


def _body(x_ref, pe_ref, o_ref):
    o_ref[...] = x_ref[...] + pe_ref[...]


def kernel(x, pe):
    b, l, d = x.shape
    nl = l // _BL
    return pl.pallas_call(
        _body,
        grid=(nl, b),
        in_specs=[
            pl.BlockSpec((1, _BL, d), lambda i, j: (j, i, 0)),
            pl.BlockSpec((_BL, d), lambda i, j: (i, 0)),
        ],
        out_specs=pl.BlockSpec((1, _BL, d), lambda i, j: (j, i, 0)),
        out_shape=jax.ShapeDtypeStruct((b, l, d), x.dtype),
    )(x, pe)


# speedup vs baseline: 1.5568x; 1.0782x over previous
import jax
import jax.numpy as jnp
from jax.experimental import pallas as pl


_BL = 2048


def _body(x_ref, pe_ref, o_ref):
    o_ref[...] = x_ref[...] + pe_ref[...]


def kernel(x, pe):
    b, l, d = x.shape
    nl = l // _BL
    return pl.pallas_call(
        _body,
        grid=(nl, b),
        in_specs=[
            pl.BlockSpec((1, _BL, d), lambda i, j: (j, i, 0)),
            pl.BlockSpec((_BL, d), lambda i, j: (i, 0)),
        ],
        out_specs=pl.BlockSpec((1, _BL, d), lambda i, j: (j, i, 0)),
        out_shape=jax.ShapeDtypeStruct((b, l, d), x.dtype),
    )(x, pe)
